# baseline (speedup 1.0000x reference)
import functools

import jax
import jax.numpy as jnp
from jax import lax
from jax.experimental import pallas as pl
from jax.experimental.pallas import tpu as pltpu
from jax.experimental.pallas import tpu_sc as plsc

N_NODES = 10000
IN_DIM = 128
OUT_DIM = 128
NUM_REL = 8
NUM_BASES = 4
N_EDGES = 160000

NC = 2
NS = 16
CHUNK = 128
CPW = 40
E_PAD = NC * NS * CPW * CHUNK
ACC_ROWS = 10240
ZROWS = 640

BN = 1000


ECB = N_EDGES // 10 // CHUNK
PCB = E_PAD // 10 // CHUNK - ECB


def _mm_body(coeff_ref, basis_ref, x_ref, wselfT_ref, ei_ref, et_ref,
             z_ref, out0_ref, gidx_ref, dst_ref, wfull_ref):
    @pl.when(pl.program_id(0) == 0)
    def _():
        for r in range(NUM_REL):
            acc = coeff_ref[r, 0] * basis_ref[0]
            for b in range(1, NUM_BASES):
                acc = acc + coeff_ref[r, b] * basis_ref[b]
            wfull_ref[:, r * OUT_DIM:(r + 1) * OUT_DIM] = acc

    xb = x_ref[...]
    for r in range(NUM_REL):
        z_ref[r] = jnp.dot(xb, wfull_ref[:, r * OUT_DIM:(r + 1) * OUT_DIM],
                           preferred_element_type=jnp.float32)
    out0_ref[...] = jnp.dot(xb, wselfT_ref[...], preferred_element_type=jnp.float32)

    iot = (lax.broadcasted_iota(jnp.int32, (PCB, CHUNK), 0) * CHUNK
           + lax.broadcasted_iota(jnp.int32, (PCB, CHUNK), 1))
    gidx_ref[...] = jnp.concatenate(
        [et_ref[0] * N_NODES + ei_ref[0, 0], iot], axis=0)
    dst_ref[...] = jnp.concatenate(
        [ei_ref[1, 0], N_NODES + iot % (ACC_ROWS - N_NODES)], axis=0)


def _fin_body(out0_ref, p0_ref, p1_ref, bias_ref, o_ref):
    o_ref[...] = jnp.maximum(
        out0_ref[...] + p0_ref[0] + p1_ref[0] + bias_ref[...], 0.0)


def _sc_body(z_hbm, gidx_hbm, dst_hbm, zero_hbm, out_hbm,
             gidx_v, dst_v, rows0, rows1, acc_sh, gsem0, gsem1, ssem0, ssem1):
    c = lax.axis_index("c")
    s = lax.axis_index("s")
    pltpu.sync_copy(zero_hbm, acc_sh.at[pl.ds(s * ZROWS, ZROWS)])
    row0 = (c * NS + s) * CPW
    pltpu.sync_copy(gidx_hbm.at[pl.ds(row0, CPW)], gidx_v)
    pltpu.sync_copy(dst_hbm.at[pl.ds(row0, CPW)], dst_v)
    plsc.subcore_barrier()

    def wait_gather(buf, sem):
        pltpu.make_async_copy(z_hbm.at[gidx_v.at[0]], buf, sem).wait()

    def wait_scatter(buf, sem):
        pltpu.make_async_copy(buf, acc_sh.at[dst_v.at[0]], sem).wait()

    pltpu.async_copy(z_hbm.at[gidx_v.at[0]], rows0, gsem0)
    pltpu.async_copy(z_hbm.at[gidx_v.at[1]], rows1, gsem1)

    def body(k, carry):
        j0 = 2 * k
        wait_gather(rows0, gsem0)
        pltpu.async_copy(rows0, acc_sh.at[dst_v.at[j0]], ssem0, add=True)
        wait_gather(rows1, gsem1)
        pltpu.async_copy(rows1, acc_sh.at[dst_v.at[j0 + 1]], ssem1, add=True)
        nxt0 = jnp.minimum(j0 + 2, CPW - 1)
        nxt1 = jnp.minimum(j0 + 3, CPW - 1)
        wait_scatter(rows0, ssem0)
        pltpu.async_copy(z_hbm.at[gidx_v.at[nxt0]], rows0, gsem0)
        wait_scatter(rows1, ssem1)
        pltpu.async_copy(z_hbm.at[gidx_v.at[nxt1]], rows1, gsem1)
        return carry

    lax.fori_loop(0, CPW // 2, body, 0)
    wait_gather(rows0, gsem0)
    wait_gather(rows1, gsem1)
    plsc.subcore_barrier()
    r = s * ZROWS
    pltpu.sync_copy(acc_sh.at[pl.ds(r, ZROWS)],
                    out_hbm.at[pl.ds(c * ACC_ROWS + r, ZROWS)])


def kernel(x, edge_index, edge_type, basis_weights, coeff, W_self, bias):
    f32 = jnp.float32

    mm = pl.pallas_call(
        _mm_body,
        grid=(N_NODES // BN,),
        in_specs=[
            pl.BlockSpec(memory_space=pltpu.SMEM),
            pl.BlockSpec((NUM_BASES, IN_DIM, OUT_DIM), lambda i: (0, 0, 0)),
            pl.BlockSpec((BN, IN_DIM), lambda i: (i, 0)),
            pl.BlockSpec((IN_DIM, OUT_DIM), lambda i: (0, 0)),
            pl.BlockSpec((2, 1, ECB, CHUNK), lambda i: (0, i, 0, 0)),
            pl.BlockSpec((1, ECB, CHUNK), lambda i: (i, 0, 0)),
        ],
        out_specs=[
            pl.BlockSpec((NUM_REL, BN, OUT_DIM), lambda i: (0, i, 0)),
            pl.BlockSpec((BN, OUT_DIM), lambda i: (i, 0)),
            pl.BlockSpec((ECB + PCB, CHUNK), lambda i: (i, 0)),
            pl.BlockSpec((ECB + PCB, CHUNK), lambda i: (i, 0)),
        ],
        out_shape=[
            jax.ShapeDtypeStruct((NUM_REL, N_NODES, OUT_DIM), f32),
            jax.ShapeDtypeStruct((N_NODES, OUT_DIM), f32),
            jax.ShapeDtypeStruct((E_PAD // CHUNK, CHUNK), jnp.int32),
            jax.ShapeDtypeStruct((E_PAD // CHUNK, CHUNK), jnp.int32),
        ],
        scratch_shapes=[pltpu.VMEM((IN_DIM, NUM_REL * OUT_DIM), f32)],
    )
    z, out0, gidx2d, dst2d = mm(
        coeff, basis_weights, x, W_self.T,
        edge_index.reshape(2, 10, ECB, CHUNK),
        edge_type.reshape(10, ECB, CHUNK))
    z_rows = z.reshape(N_NODES * NUM_REL, OUT_DIM)
    zero_blk = jnp.zeros((ZROWS, OUT_DIM), f32)

    mesh = plsc.VectorSubcoreMesh(core_axis_name="c", subcore_axis_name="s",
                                  num_cores=NC, num_subcores=NS)
    sc = functools.partial(
        pl.kernel,
        out_type=jax.ShapeDtypeStruct((NC * ACC_ROWS, OUT_DIM), f32),
        mesh=mesh,
        scratch_types=[
            pltpu.VMEM((CPW, CHUNK), jnp.int32),
            pltpu.VMEM((CPW, CHUNK), jnp.int32),
            pltpu.VMEM((CHUNK, OUT_DIM), f32),
            pltpu.VMEM((CHUNK, OUT_DIM), f32),
            pltpu.VMEM_SHARED((ACC_ROWS, OUT_DIM), f32),
            pltpu.SemaphoreType.DMA,
            pltpu.SemaphoreType.DMA,
            pltpu.SemaphoreType.DMA,
            pltpu.SemaphoreType.DMA,
        ],
    )(_sc_body)
    partials = sc(z_rows, gidx2d, dst2d, zero_blk)

    p3 = partials.reshape(NC, ACC_ROWS, OUT_DIM)
    BF = 400
    fin = pl.pallas_call(
        _fin_body,
        grid=(N_NODES // BF,),
        in_specs=[
            pl.BlockSpec((BF, OUT_DIM), lambda i: (i, 0)),
            pl.BlockSpec((1, BF, OUT_DIM), lambda i: (0, i, 0)),
            pl.BlockSpec((1, BF, OUT_DIM), lambda i: (1, i, 0)),
            pl.BlockSpec((1, OUT_DIM), lambda i: (0, 0)),
        ],
        out_specs=pl.BlockSpec((BF, OUT_DIM), lambda i: (i, 0)),
        out_shape=jax.ShapeDtypeStruct((N_NODES, OUT_DIM), f32),
    )
    return fin(out0, p3, p3, bias[None, :])


# speedup vs baseline: 1.0548x; 1.0548x over previous
import functools

import jax
import jax.numpy as jnp
from jax import lax
from jax.experimental import pallas as pl
from jax.experimental.pallas import tpu as pltpu
from jax.experimental.pallas import tpu_sc as plsc

N_NODES = 10000
IN_DIM = 128
OUT_DIM = 128
NUM_REL = 8
NUM_BASES = 4
N_EDGES = 160000

NC = 2
NS = 16
CHUNK = 128
CPW = 40
E_PAD = NC * NS * CPW * CHUNK
ACC_ROWS = 10240
ZROWS = 640

BN = 1000


ECB = N_EDGES // 10 // CHUNK
PCB = E_PAD // 10 // CHUNK - ECB


def _mm_body(coeff_ref, basis_ref, x_ref, wselfT_ref, ei_ref, et_ref,
             z_ref, out0_ref, gidx_ref, dst_ref, wfull_ref):
    @pl.when(pl.program_id(0) == 0)
    def _():
        for r in range(NUM_REL):
            acc = coeff_ref[r, 0] * basis_ref[0]
            for b in range(1, NUM_BASES):
                acc = acc + coeff_ref[r, b] * basis_ref[b]
            wfull_ref[:, r * OUT_DIM:(r + 1) * OUT_DIM] = acc

    xb = x_ref[...]
    for r in range(NUM_REL):
        z_ref[r] = jnp.dot(xb, wfull_ref[:, r * OUT_DIM:(r + 1) * OUT_DIM],
                           preferred_element_type=jnp.float32)
    out0_ref[...] = jnp.dot(xb, wselfT_ref[...], preferred_element_type=jnp.float32)

    iot = (lax.broadcasted_iota(jnp.int32, (PCB, CHUNK), 0) * CHUNK
           + lax.broadcasted_iota(jnp.int32, (PCB, CHUNK), 1))
    src2 = ei_ref[0].reshape(ECB, CHUNK)
    dstb = ei_ref[1].reshape(ECB, CHUNK)
    et2 = et_ref[0].reshape(ECB, CHUNK)
    gidx_ref[...] = jnp.concatenate([et2 * N_NODES + src2, iot], axis=0)
    dst_ref[...] = jnp.concatenate(
        [dstb, N_NODES + iot % (ACC_ROWS - N_NODES)], axis=0)


def _fin_body(out0_ref, p0_ref, p1_ref, bias_ref, o_ref):
    o_ref[...] = jnp.maximum(
        out0_ref[...] + p0_ref[0] + p1_ref[0] + bias_ref[...], 0.0)


def _sc_body(z_hbm, gidx_hbm, dst_hbm, zero_hbm, out_hbm,
             gidx_v, dst_v, rows0, rows1, acc_sh, gsem0, gsem1, ssem0, ssem1):
    c = lax.axis_index("c")
    s = lax.axis_index("s")
    pltpu.sync_copy(zero_hbm, acc_sh.at[pl.ds(s * ZROWS, ZROWS)])
    row0 = (c * NS + s) * CPW
    pltpu.sync_copy(gidx_hbm.at[pl.ds(row0, CPW)], gidx_v)
    pltpu.sync_copy(dst_hbm.at[pl.ds(row0, CPW)], dst_v)
    plsc.subcore_barrier()

    def wait_gather(buf, sem):
        pltpu.make_async_copy(z_hbm.at[gidx_v.at[0]], buf, sem).wait()

    def wait_scatter(buf, sem):
        pltpu.make_async_copy(buf, acc_sh.at[dst_v.at[0]], sem).wait()

    pltpu.async_copy(z_hbm.at[gidx_v.at[0]], rows0, gsem0)
    pltpu.async_copy(z_hbm.at[gidx_v.at[1]], rows1, gsem1)

    def body(k, carry):
        j0 = 2 * k
        wait_gather(rows0, gsem0)
        pltpu.async_copy(rows0, acc_sh.at[dst_v.at[j0]], ssem0, add=True)
        wait_gather(rows1, gsem1)
        pltpu.async_copy(rows1, acc_sh.at[dst_v.at[j0 + 1]], ssem1, add=True)
        nxt0 = jnp.minimum(j0 + 2, CPW - 1)
        nxt1 = jnp.minimum(j0 + 3, CPW - 1)
        wait_scatter(rows0, ssem0)
        pltpu.async_copy(z_hbm.at[gidx_v.at[nxt0]], rows0, gsem0)
        wait_scatter(rows1, ssem1)
        pltpu.async_copy(z_hbm.at[gidx_v.at[nxt1]], rows1, gsem1)
        return carry

    lax.fori_loop(0, CPW // 2, body, 0)
    wait_gather(rows0, gsem0)
    wait_gather(rows1, gsem1)
    plsc.subcore_barrier()
    r = s * ZROWS
    pltpu.sync_copy(acc_sh.at[pl.ds(r, ZROWS)],
                    out_hbm.at[pl.ds(c * ACC_ROWS + r, ZROWS)])


def kernel(x, edge_index, edge_type, basis_weights, coeff, W_self, bias):
    f32 = jnp.float32

    mm = pl.pallas_call(
        _mm_body,
        grid=(N_NODES // BN,),
        in_specs=[
            pl.BlockSpec(memory_space=pltpu.SMEM),
            pl.BlockSpec((NUM_BASES, IN_DIM, OUT_DIM), lambda i: (0, 0, 0)),
            pl.BlockSpec((BN, IN_DIM), lambda i: (i, 0)),
            pl.BlockSpec((IN_DIM, OUT_DIM), lambda i: (0, 0)),
            pl.BlockSpec((2, ECB * CHUNK), lambda i: (0, i)),
            pl.BlockSpec((1, ECB * CHUNK), lambda i: (0, i)),
        ],
        out_specs=[
            pl.BlockSpec((NUM_REL, BN, OUT_DIM), lambda i: (0, i, 0)),
            pl.BlockSpec((BN, OUT_DIM), lambda i: (i, 0)),
            pl.BlockSpec((ECB + PCB, CHUNK), lambda i: (i, 0)),
            pl.BlockSpec((ECB + PCB, CHUNK), lambda i: (i, 0)),
        ],
        out_shape=[
            jax.ShapeDtypeStruct((NUM_REL, N_NODES, OUT_DIM), f32),
            jax.ShapeDtypeStruct((N_NODES, OUT_DIM), f32),
            jax.ShapeDtypeStruct((E_PAD // CHUNK, CHUNK), jnp.int32),
            jax.ShapeDtypeStruct((E_PAD // CHUNK, CHUNK), jnp.int32),
        ],
        scratch_shapes=[pltpu.VMEM((IN_DIM, NUM_REL * OUT_DIM), f32)],
    )
    z, out0, gidx2d, dst2d = mm(
        coeff, basis_weights, x, W_self.T, edge_index,
        edge_type.reshape(1, N_EDGES))
    z_rows = z.reshape(N_NODES * NUM_REL, OUT_DIM)
    zero_blk = jnp.zeros((ZROWS, OUT_DIM), f32)

    mesh = plsc.VectorSubcoreMesh(core_axis_name="c", subcore_axis_name="s",
                                  num_cores=NC, num_subcores=NS)
    sc = functools.partial(
        pl.kernel,
        out_type=jax.ShapeDtypeStruct((NC * ACC_ROWS, OUT_DIM), f32),
        mesh=mesh,
        scratch_types=[
            pltpu.VMEM((CPW, CHUNK), jnp.int32),
            pltpu.VMEM((CPW, CHUNK), jnp.int32),
            pltpu.VMEM((CHUNK, OUT_DIM), f32),
            pltpu.VMEM((CHUNK, OUT_DIM), f32),
            pltpu.VMEM_SHARED((ACC_ROWS, OUT_DIM), f32),
            pltpu.SemaphoreType.DMA,
            pltpu.SemaphoreType.DMA,
            pltpu.SemaphoreType.DMA,
            pltpu.SemaphoreType.DMA,
        ],
    )(_sc_body)
    partials = sc(z_rows, gidx2d, dst2d, zero_blk)

    p3 = partials.reshape(NC, ACC_ROWS, OUT_DIM)
    BF = 1000
    fin = pl.pallas_call(
        _fin_body,
        grid=(N_NODES // BF,),
        in_specs=[
            pl.BlockSpec((BF, OUT_DIM), lambda i: (i, 0)),
            pl.BlockSpec((1, BF, OUT_DIM), lambda i: (0, i, 0)),
            pl.BlockSpec((1, BF, OUT_DIM), lambda i: (1, i, 0)),
            pl.BlockSpec((1, OUT_DIM), lambda i: (0, 0)),
        ],
        out_specs=pl.BlockSpec((BF, OUT_DIM), lambda i: (i, 0)),
        out_shape=jax.ShapeDtypeStruct((N_NODES, OUT_DIM), f32),
    )
    return fin(out0, p3, p3, bias[None, :])


# speedup vs baseline: 1.0774x; 1.0214x over previous
import functools

import jax
import jax.numpy as jnp
from jax import lax
from jax.experimental import pallas as pl
from jax.experimental.pallas import tpu as pltpu
from jax.experimental.pallas import tpu_sc as plsc

N_NODES = 10000
IN_DIM = 128
OUT_DIM = 128
NUM_REL = 8
NUM_BASES = 4
N_EDGES = 160000

NC = 2
NS = 16
CHUNK = 128
CPW = 40
E_PAD = NC * NS * CPW * CHUNK
ACC_ROWS = 10240
ZROWS = 640

BN = 1000


ECB = N_EDGES // 10 // CHUNK
PCB = E_PAD // 10 // CHUNK - ECB


def _mm_body(coeff_ref, basis_ref, x_ref, wselfT_ref, ei_ref, et_ref,
             z_ref, out0_ref, gidx_ref, dst_ref, wfull_ref):
    @pl.when(pl.program_id(0) == 0)
    def _():
        for r in range(NUM_REL):
            acc = coeff_ref[r, 0] * basis_ref[0]
            for b in range(1, NUM_BASES):
                acc = acc + coeff_ref[r, b] * basis_ref[b]
            wfull_ref[:, r * OUT_DIM:(r + 1) * OUT_DIM] = acc

    xb = x_ref[...]
    y = jnp.dot(xb, wfull_ref[...], preferred_element_type=jnp.float32)
    for r in range(NUM_REL):
        z_ref[r] = y[:, r * OUT_DIM:(r + 1) * OUT_DIM]
    out0_ref[...] = jnp.dot(xb, wselfT_ref[...], preferred_element_type=jnp.float32)

    iot = (lax.broadcasted_iota(jnp.int32, (PCB, CHUNK), 0) * CHUNK
           + lax.broadcasted_iota(jnp.int32, (PCB, CHUNK), 1))
    src2 = ei_ref[0].reshape(ECB, CHUNK)
    dstb = ei_ref[1].reshape(ECB, CHUNK)
    et2 = et_ref[0].reshape(ECB, CHUNK)
    gidx_ref[...] = jnp.concatenate([et2 * N_NODES + src2, iot], axis=0)
    dst_ref[...] = jnp.concatenate(
        [dstb, N_NODES + iot % (ACC_ROWS - N_NODES)], axis=0)


def _fin_body(out0_ref, p0_ref, p1_ref, bias_ref, o_ref):
    o_ref[...] = jnp.maximum(
        out0_ref[...] + p0_ref[0] + p1_ref[0] + bias_ref[...], 0.0)


def _sc_body(z_hbm, gidx_hbm, dst_hbm, zero_hbm, out_hbm,
             gidx_v, dst_v, rows0, rows1, acc_sh, gsem0, gsem1, ssem0, ssem1):
    c = lax.axis_index("c")
    s = lax.axis_index("s")
    row0 = (c * NS + s) * CPW
    z0 = pltpu.async_copy(zero_hbm, acc_sh.at[pl.ds(s * ZROWS, ZROWS)], gsem0)
    i0 = pltpu.async_copy(gidx_hbm.at[pl.ds(row0, CPW)], gidx_v, gsem1)
    i1 = pltpu.async_copy(dst_hbm.at[pl.ds(row0, CPW)], dst_v, ssem0)
    z0.wait()
    i0.wait()
    i1.wait()
    plsc.subcore_barrier()

    def wait_gather(buf, sem):
        pltpu.make_async_copy(z_hbm.at[gidx_v.at[0]], buf, sem).wait()

    def wait_scatter(buf, sem):
        pltpu.make_async_copy(buf, acc_sh.at[dst_v.at[0]], sem).wait()

    pltpu.async_copy(z_hbm.at[gidx_v.at[0]], rows0, gsem0)
    pltpu.async_copy(z_hbm.at[gidx_v.at[1]], rows1, gsem1)

    def body(k, carry):
        j0 = 2 * k
        wait_gather(rows0, gsem0)
        pltpu.async_copy(rows0, acc_sh.at[dst_v.at[j0]], ssem0, add=True)
        wait_gather(rows1, gsem1)
        pltpu.async_copy(rows1, acc_sh.at[dst_v.at[j0 + 1]], ssem1, add=True)
        nxt0 = jnp.minimum(j0 + 2, CPW - 1)
        nxt1 = jnp.minimum(j0 + 3, CPW - 1)
        wait_scatter(rows0, ssem0)
        pltpu.async_copy(z_hbm.at[gidx_v.at[nxt0]], rows0, gsem0)
        wait_scatter(rows1, ssem1)
        pltpu.async_copy(z_hbm.at[gidx_v.at[nxt1]], rows1, gsem1)
        return carry

    lax.fori_loop(0, CPW // 2, body, 0)
    wait_gather(rows0, gsem0)
    wait_gather(rows1, gsem1)
    plsc.subcore_barrier()
    r = s * ZROWS
    pltpu.sync_copy(acc_sh.at[pl.ds(r, ZROWS)],
                    out_hbm.at[pl.ds(c * ACC_ROWS + r, ZROWS)])


def kernel(x, edge_index, edge_type, basis_weights, coeff, W_self, bias):
    f32 = jnp.float32

    mm = pl.pallas_call(
        _mm_body,
        grid=(N_NODES // BN,),
        in_specs=[
            pl.BlockSpec(memory_space=pltpu.SMEM),
            pl.BlockSpec((NUM_BASES, IN_DIM, OUT_DIM), lambda i: (0, 0, 0)),
            pl.BlockSpec((BN, IN_DIM), lambda i: (i, 0)),
            pl.BlockSpec((IN_DIM, OUT_DIM), lambda i: (0, 0)),
            pl.BlockSpec((2, ECB * CHUNK), lambda i: (0, i)),
            pl.BlockSpec((1, ECB * CHUNK), lambda i: (0, i)),
        ],
        out_specs=[
            pl.BlockSpec((NUM_REL, BN, OUT_DIM), lambda i: (0, i, 0)),
            pl.BlockSpec((BN, OUT_DIM), lambda i: (i, 0)),
            pl.BlockSpec((ECB + PCB, CHUNK), lambda i: (i, 0)),
            pl.BlockSpec((ECB + PCB, CHUNK), lambda i: (i, 0)),
        ],
        out_shape=[
            jax.ShapeDtypeStruct((NUM_REL, N_NODES, OUT_DIM), f32),
            jax.ShapeDtypeStruct((N_NODES, OUT_DIM), f32),
            jax.ShapeDtypeStruct((E_PAD // CHUNK, CHUNK), jnp.int32),
            jax.ShapeDtypeStruct((E_PAD // CHUNK, CHUNK), jnp.int32),
        ],
        scratch_shapes=[pltpu.VMEM((IN_DIM, NUM_REL * OUT_DIM), f32)],
    )
    z, out0, gidx2d, dst2d = mm(
        coeff, basis_weights, x, W_self.T, edge_index,
        edge_type.reshape(1, N_EDGES))
    z_rows = z.reshape(N_NODES * NUM_REL, OUT_DIM)
    zero_blk = jnp.zeros((ZROWS, OUT_DIM), f32)

    mesh = plsc.VectorSubcoreMesh(core_axis_name="c", subcore_axis_name="s",
                                  num_cores=NC, num_subcores=NS)
    sc = functools.partial(
        pl.kernel,
        out_type=jax.ShapeDtypeStruct((NC * ACC_ROWS, OUT_DIM), f32),
        mesh=mesh,
        scratch_types=[
            pltpu.VMEM((CPW, CHUNK), jnp.int32),
            pltpu.VMEM((CPW, CHUNK), jnp.int32),
            pltpu.VMEM((CHUNK, OUT_DIM), f32),
            pltpu.VMEM((CHUNK, OUT_DIM), f32),
            pltpu.VMEM_SHARED((ACC_ROWS, OUT_DIM), f32),
            pltpu.SemaphoreType.DMA,
            pltpu.SemaphoreType.DMA,
            pltpu.SemaphoreType.DMA,
            pltpu.SemaphoreType.DMA,
        ],
    )(_sc_body)
    partials = sc(z_rows, gidx2d, dst2d, zero_blk)

    p3 = partials.reshape(NC, ACC_ROWS, OUT_DIM)
    BF = 1000
    fin = pl.pallas_call(
        _fin_body,
        grid=(N_NODES // BF,),
        in_specs=[
            pl.BlockSpec((BF, OUT_DIM), lambda i: (i, 0)),
            pl.BlockSpec((1, BF, OUT_DIM), lambda i: (0, i, 0)),
            pl.BlockSpec((1, BF, OUT_DIM), lambda i: (1, i, 0)),
            pl.BlockSpec((1, OUT_DIM), lambda i: (0, 0)),
        ],
        out_specs=pl.BlockSpec((BF, OUT_DIM), lambda i: (i, 0)),
        out_shape=jax.ShapeDtypeStruct((N_NODES, OUT_DIM), f32),
    )
    return fin(out0, p3, p3, bias[None, :])


# speedup vs baseline: 1.1068x; 1.0274x over previous
import functools

import jax
import jax.numpy as jnp
from jax import lax
from jax.experimental import pallas as pl
from jax.experimental.pallas import tpu as pltpu
from jax.experimental.pallas import tpu_sc as plsc

N_NODES = 10000
IN_DIM = 128
OUT_DIM = 128
NUM_REL = 8
NUM_BASES = 4
N_EDGES = 160000

NC = 2
NS = 16
CHUNK = 128
CPW = 40
E_PAD = NC * NS * CPW * CHUNK
ACC_ROWS = 10240
ZROWS = 640

BN = 1000


ECB = N_EDGES // 10 // CHUNK
PCB = E_PAD // 10 // CHUNK - ECB


def _mm_body(coeff_ref, basis_ref, x_ref, wselfT_ref, ei_ref, et_ref,
             z_ref, out0_ref, gidx_ref, dst_ref, wfull_ref):
    @pl.when(pl.program_id(0) == 0)
    def _():
        for r in range(NUM_REL):
            acc = coeff_ref[r, 0] * basis_ref[0]
            for b in range(1, NUM_BASES):
                acc = acc + coeff_ref[r, b] * basis_ref[b]
            wfull_ref[:, r * OUT_DIM:(r + 1) * OUT_DIM] = acc

    xb = x_ref[...]
    y = jnp.dot(xb, wfull_ref[...], preferred_element_type=jnp.float32)
    for r in range(NUM_REL):
        z_ref[r] = y[:, r * OUT_DIM:(r + 1) * OUT_DIM]
    out0_ref[...] = jnp.dot(xb, wselfT_ref[...], preferred_element_type=jnp.float32)

    iot = (lax.broadcasted_iota(jnp.int32, (PCB, CHUNK), 0) * CHUNK
           + lax.broadcasted_iota(jnp.int32, (PCB, CHUNK), 1))
    src2 = ei_ref[0].reshape(ECB, CHUNK)
    dstb = ei_ref[1].reshape(ECB, CHUNK)
    et2 = et_ref[pl.ds(pl.program_id(0) * ECB * CHUNK,
                       ECB * CHUNK)].reshape(ECB, CHUNK)
    gidx_ref[...] = jnp.concatenate([et2 * N_NODES + src2, iot], axis=0)
    dst_ref[...] = jnp.concatenate(
        [dstb, N_NODES + iot % (ACC_ROWS - N_NODES)], axis=0)


def _fin_body(out0_ref, p0_ref, p1_ref, bias_ref, o_ref):
    o_ref[...] = jnp.maximum(
        out0_ref[...] + p0_ref[0] + p1_ref[0] + bias_ref[...], 0.0)


def _sc_body(z_hbm, gidx_hbm, dst_hbm, zero_hbm, out_hbm,
             gidx_v, dst_v, rows0, rows1, acc_sh, gsem0, gsem1, ssem0, ssem1):
    c = lax.axis_index("c")
    s = lax.axis_index("s")
    row0 = (c * NS + s) * CPW
    z0 = pltpu.async_copy(zero_hbm, acc_sh.at[pl.ds(s * ZROWS, ZROWS)], gsem0)
    i0 = pltpu.async_copy(gidx_hbm.at[pl.ds(row0, CPW)], gidx_v, gsem1)
    i1 = pltpu.async_copy(dst_hbm.at[pl.ds(row0, CPW)], dst_v, ssem0)
    z0.wait()
    i0.wait()
    i1.wait()
    plsc.subcore_barrier()

    def wait_gather(buf, sem):
        pltpu.make_async_copy(z_hbm.at[gidx_v.at[0]], buf, sem).wait()

    def wait_scatter(buf, sem):
        pltpu.make_async_copy(buf, acc_sh.at[dst_v.at[0]], sem).wait()

    pltpu.async_copy(z_hbm.at[gidx_v.at[0]], rows0, gsem0)
    pltpu.async_copy(z_hbm.at[gidx_v.at[1]], rows1, gsem1)

    def body(k, carry):
        j0 = 2 * k
        wait_gather(rows0, gsem0)
        pltpu.async_copy(rows0, acc_sh.at[dst_v.at[j0]], ssem0, add=True)
        wait_gather(rows1, gsem1)
        pltpu.async_copy(rows1, acc_sh.at[dst_v.at[j0 + 1]], ssem1, add=True)
        nxt0 = jnp.minimum(j0 + 2, CPW - 1)
        nxt1 = jnp.minimum(j0 + 3, CPW - 1)
        wait_scatter(rows0, ssem0)
        pltpu.async_copy(z_hbm.at[gidx_v.at[nxt0]], rows0, gsem0)
        wait_scatter(rows1, ssem1)
        pltpu.async_copy(z_hbm.at[gidx_v.at[nxt1]], rows1, gsem1)
        return carry

    lax.fori_loop(0, CPW // 2, body, 0)
    wait_gather(rows0, gsem0)
    wait_gather(rows1, gsem1)
    plsc.subcore_barrier()
    r = s * ZROWS
    pltpu.sync_copy(acc_sh.at[pl.ds(r, ZROWS)],
                    out_hbm.at[pl.ds(c * ACC_ROWS + r, ZROWS)])


def kernel(x, edge_index, edge_type, basis_weights, coeff, W_self, bias):
    f32 = jnp.float32

    mm = pl.pallas_call(
        _mm_body,
        grid=(N_NODES // BN,),
        in_specs=[
            pl.BlockSpec(memory_space=pltpu.SMEM),
            pl.BlockSpec((NUM_BASES, IN_DIM, OUT_DIM), lambda i: (0, 0, 0)),
            pl.BlockSpec((BN, IN_DIM), lambda i: (i, 0)),
            pl.BlockSpec((IN_DIM, OUT_DIM), lambda i: (0, 0)),
            pl.BlockSpec((2, ECB * CHUNK), lambda i: (0, i)),
            pl.BlockSpec((N_EDGES,), lambda i: (0,)),
        ],
        out_specs=[
            pl.BlockSpec((NUM_REL, BN, OUT_DIM), lambda i: (0, i, 0)),
            pl.BlockSpec((BN, OUT_DIM), lambda i: (i, 0)),
            pl.BlockSpec((ECB + PCB, CHUNK), lambda i: (i, 0)),
            pl.BlockSpec((ECB + PCB, CHUNK), lambda i: (i, 0)),
        ],
        out_shape=[
            jax.ShapeDtypeStruct((NUM_REL, N_NODES, OUT_DIM), f32),
            jax.ShapeDtypeStruct((N_NODES, OUT_DIM), f32),
            jax.ShapeDtypeStruct((E_PAD // CHUNK, CHUNK), jnp.int32),
            jax.ShapeDtypeStruct((E_PAD // CHUNK, CHUNK), jnp.int32),
        ],
        scratch_shapes=[pltpu.VMEM((IN_DIM, NUM_REL * OUT_DIM), f32)],
    )
    z, out0, gidx2d, dst2d = mm(
        coeff, basis_weights, x, W_self.T, edge_index, edge_type)
    z_rows = z.reshape(N_NODES * NUM_REL, OUT_DIM)
    zero_blk = jnp.zeros((ZROWS, OUT_DIM), f32)

    mesh = plsc.VectorSubcoreMesh(core_axis_name="c", subcore_axis_name="s",
                                  num_cores=NC, num_subcores=NS)
    sc = functools.partial(
        pl.kernel,
        out_type=jax.ShapeDtypeStruct((NC * ACC_ROWS, OUT_DIM), f32),
        mesh=mesh,
        scratch_types=[
            pltpu.VMEM((CPW, CHUNK), jnp.int32),
            pltpu.VMEM((CPW, CHUNK), jnp.int32),
            pltpu.VMEM((CHUNK, OUT_DIM), f32),
            pltpu.VMEM((CHUNK, OUT_DIM), f32),
            pltpu.VMEM_SHARED((ACC_ROWS, OUT_DIM), f32),
            pltpu.SemaphoreType.DMA,
            pltpu.SemaphoreType.DMA,
            pltpu.SemaphoreType.DMA,
            pltpu.SemaphoreType.DMA,
        ],
    )(_sc_body)
    partials = sc(z_rows, gidx2d, dst2d, zero_blk)

    p3 = partials.reshape(NC, ACC_ROWS, OUT_DIM)
    BF = 1000
    fin = pl.pallas_call(
        _fin_body,
        grid=(N_NODES // BF,),
        in_specs=[
            pl.BlockSpec((BF, OUT_DIM), lambda i: (i, 0)),
            pl.BlockSpec((1, BF, OUT_DIM), lambda i: (0, i, 0)),
            pl.BlockSpec((1, BF, OUT_DIM), lambda i: (1, i, 0)),
            pl.BlockSpec((1, OUT_DIM), lambda i: (0, 0)),
        ],
        out_specs=pl.BlockSpec((BF, OUT_DIM), lambda i: (i, 0)),
        out_shape=jax.ShapeDtypeStruct((N_NODES, OUT_DIM), f32),
    )
    return fin(out0, p3, p3, bias[None, :])
